# baseline (speedup 1.0000x reference)
import functools

import jax
import jax.numpy as jnp
from jax import lax
from jax.experimental import pallas as pl
from jax.experimental.pallas import tpu as pltpu
from jax.experimental.pallas import tpu_sc as plsc

_CB = 8192
_D = 256
_M_BLK = 1024
_N_BLK = 512
_N_STEPS = _CB // _N_BLK


def _argmin_body(h_ref, cb_ref, idx_ref, dsum_ref, best_val, best_idx):
    n = pl.program_id(1)
    hm = h_ref[...]
    c = cb_ref[...]
    mm = lax.dot_general(hm, c, (((1,), (1,)), ((), ())),
                         preferred_element_type=jnp.float32)
    h2 = jnp.sum(hm * hm, axis=1, keepdims=True)
    c2 = jnp.sum(c * c, axis=1)
    dist = (h2 + c2[None, :]) - 2.0 * mm
    tile_min = jnp.min(dist, axis=1, keepdims=True)
    cols = lax.broadcasted_iota(jnp.int32, dist.shape, 1) + n * _N_BLK
    tile_idx = jnp.min(jnp.where(dist == tile_min, cols, _CB), axis=1,
                       keepdims=True)

    @pl.when(n == 0)
    def _init():
        best_val[...] = tile_min
        best_idx[...] = tile_idx

    @pl.when(n > 0)
    def _update():
        bv = best_val[...]
        better = tile_min < bv
        best_val[...] = jnp.where(better, tile_min, bv)
        best_idx[...] = jnp.where(better, tile_idx, best_idx[...])

    @pl.when(n == _N_STEPS - 1)
    def _finish():
        idx_ref[...] = best_idx[...]

        @pl.when(pl.program_id(0) == 0)
        def _zero():
            dsum_ref[...] = jnp.zeros_like(dsum_ref)

        dsum_ref[...] += jnp.sum(best_val[...], keepdims=True)


def _argmin_call(flat, codebook):
    m_steps = flat.shape[0] // _M_BLK
    return pl.pallas_call(
        _argmin_body,
        grid=(m_steps, _N_STEPS),
        in_specs=[
            pl.BlockSpec((_M_BLK, _D), lambda m, n: (m, 0)),
            pl.BlockSpec((_N_BLK, _D), lambda m, n: (n, 0)),
        ],
        out_specs=[
            pl.BlockSpec((_M_BLK, 1), lambda m, n: (m, 0)),
            pl.BlockSpec((1, 1), lambda m, n: (0, 0)),
        ],
        out_shape=[
            jax.ShapeDtypeStruct((flat.shape[0], 1), jnp.int32),
            jax.ShapeDtypeStruct((1, 1), jnp.float32),
        ],
        scratch_shapes=[
            pltpu.VMEM((_M_BLK, 1), jnp.float32),
            pltpu.VMEM((_M_BLK, 1), jnp.int32),
        ],
        compiler_params=pltpu.CompilerParams(
            dimension_semantics=("arbitrary", "arbitrary"),
        ),
    )(flat, codebook)


def _gather_z(codebook, idx_flat):
    info = plsc.get_sparse_core_info()
    nc, ns = info.num_cores, info.num_subcores
    nw = nc * ns
    b = idx_flat.shape[0]
    b_per_w = b // nw
    chunk = 128
    n_chunks = b_per_w // chunk
    idx2d = idx_flat.reshape(nw * n_chunks, chunk)
    mesh = plsc.VectorSubcoreMesh(core_axis_name="c", subcore_axis_name="s")

    @functools.partial(
        pl.kernel,
        out_type=jax.ShapeDtypeStruct((b, _D), jnp.float32),
        mesh=mesh,
        scratch_types=[
            pltpu.VMEM((n_chunks, chunk), jnp.int32),
            pltpu.VMEM((chunk, _D), jnp.float32),
            pltpu.SemaphoreType.DMA,
        ],
    )
    def gather_k(cb_hbm, idx_hbm, out_hbm, idx_v, rows_v, sem):
        wid = lax.axis_index("s") * nc + lax.axis_index("c")
        pltpu.sync_copy(idx_hbm.at[pl.ds(wid * n_chunks, n_chunks)], idx_v)
        for cc in range(n_chunks):
            pltpu.async_copy(cb_hbm.at[idx_v.at[cc]], rows_v, sem).wait()
            pltpu.sync_copy(
                rows_v, out_hbm.at[pl.ds(wid * b_per_w + cc * chunk, chunk)])

    return gather_k(codebook, idx2d)


def kernel(h, codebook):
    b, t, d = h.shape
    flat = h.reshape(-1, d)
    idx_col, dsum = _argmin_call(flat, codebook)
    idx_flat = idx_col.reshape(-1)
    z = _gather_z(codebook, idx_flat)
    z_q = z.reshape(b, t, d)
    loss = dsum[0, 0] / (b * t * d)
    indices = idx_flat.reshape(b, t)
    return z_q, indices, loss, loss


# speedup vs baseline: 1.5133x; 1.5133x over previous
import functools

import jax
import jax.numpy as jnp
from jax import lax
from jax.experimental import pallas as pl
from jax.experimental.pallas import tpu as pltpu
from jax.experimental.pallas import tpu_sc as plsc

_CB = 8192
_D = 256
_M_BLK = 256


def _argmin_body(h_ref, cb_ref, idx_ref, dsum_ref, c2_ref, cols_ref):
    m = pl.program_id(0)

    @pl.when(m == 0)
    def _precompute():
        c = cb_ref[...]
        c2_ref[...] = jnp.sum(c * c, axis=1)[None, :]
        cols_ref[...] = lax.broadcasted_iota(jnp.int32, cols_ref.shape, 1)

    hm = h_ref[...]
    hm2 = hm * (-2.0)
    h2 = jnp.sum(hm * hm, axis=1, keepdims=True)
    mm2 = lax.dot_general(hm2, cb_ref[...], (((1,), (1,)), ((), ())),
                          preferred_element_type=jnp.float32)
    dist = (h2 + c2_ref[...]) + mm2
    best = jnp.min(dist, axis=1, keepdims=True)
    idx = jnp.min(jnp.where(dist == best, cols_ref[...], _CB), axis=1,
                  keepdims=True)
    idx_ref[...] = idx

    @pl.when(m == 0)
    def _zero():
        dsum_ref[...] = jnp.zeros_like(dsum_ref)

    dsum_ref[...] += jnp.sum(best, keepdims=True)


def _argmin_call(flat, codebook):
    m_steps = flat.shape[0] // _M_BLK
    return pl.pallas_call(
        _argmin_body,
        grid=(m_steps,),
        in_specs=[
            pl.BlockSpec((_M_BLK, _D), lambda m: (m, 0)),
            pl.BlockSpec((_CB, _D), lambda m: (0, 0)),
        ],
        out_specs=[
            pl.BlockSpec((_M_BLK, 1), lambda m: (m, 0)),
            pl.BlockSpec((1, 1), lambda m: (0, 0)),
        ],
        out_shape=[
            jax.ShapeDtypeStruct((flat.shape[0], 1), jnp.int32),
            jax.ShapeDtypeStruct((1, 1), jnp.float32),
        ],
        scratch_shapes=[
            pltpu.VMEM((1, _CB), jnp.float32),
            pltpu.VMEM((_M_BLK, _CB), jnp.int32),
        ],
        compiler_params=pltpu.CompilerParams(
            dimension_semantics=("arbitrary",),
        ),
    )(flat, codebook)


def _gather_z(codebook, idx_flat):
    info = plsc.get_sparse_core_info()
    nc, ns = info.num_cores, info.num_subcores
    nw = nc * ns
    b = idx_flat.shape[0]
    b_per_w = b // nw
    chunk = 128
    n_chunks = b_per_w // chunk
    idx2d = idx_flat.reshape(nw * n_chunks, chunk)
    mesh = plsc.VectorSubcoreMesh(core_axis_name="c", subcore_axis_name="s")

    @functools.partial(
        pl.kernel,
        out_type=jax.ShapeDtypeStruct((b, _D), jnp.float32),
        mesh=mesh,
        scratch_types=[
            pltpu.VMEM((n_chunks, chunk), jnp.int32),
            pltpu.VMEM((chunk, _D), jnp.float32),
            pltpu.SemaphoreType.DMA,
        ],
    )
    def gather_k(cb_hbm, idx_hbm, out_hbm, idx_v, rows_v, sem):
        wid = lax.axis_index("s") * nc + lax.axis_index("c")
        pltpu.sync_copy(idx_hbm.at[pl.ds(wid * n_chunks, n_chunks)], idx_v)
        for cc in range(n_chunks):
            pltpu.async_copy(cb_hbm.at[idx_v.at[cc]], rows_v, sem).wait()
            pltpu.sync_copy(
                rows_v, out_hbm.at[pl.ds(wid * b_per_w + cc * chunk, chunk)])

    return gather_k(codebook, idx2d)


def kernel(h, codebook):
    b, t, d = h.shape
    flat = h.reshape(-1, d)
    idx_col, dsum = _argmin_call(flat, codebook)
    idx_flat = idx_col.reshape(-1)
    z = _gather_z(codebook, idx_flat)
    z_q = z.reshape(b, t, d)
    loss = dsum[0, 0] / (b * t * d)
    indices = idx_flat.reshape(b, t)
    return z_q, indices, loss, loss


# speedup vs baseline: 2.2398x; 1.4800x over previous
import functools

import jax
import jax.numpy as jnp
from jax import lax
from jax.experimental import pallas as pl
from jax.experimental.pallas import tpu as pltpu
from jax.experimental.pallas import tpu_sc as plsc

_CB = 8192
_D = 256
_M_BLK = 2048
_SLAB = 256
_N_SLABS = _CB // _SLAB


def _argmin_body(h_ref, cbt_ref, idx_ref, dsum_ref, c2_ref):
    m = pl.program_id(0)

    @pl.when(m == 0)
    def _precompute():
        c = cbt_ref[...]
        c2_ref[...] = jnp.sum(c * c, axis=0)[None, :]

    hm = h_ref[...]
    hm2 = hm * (-2.0)
    h2 = jnp.sum(hm * hm, axis=1, keepdims=True)
    h2b = jnp.broadcast_to(h2, (_M_BLK, 128))

    val = None
    gidx = None
    for g in range(_N_SLABS):
        mm2 = lax.dot_general(
            hm2, cbt_ref[:, g * _SLAB:(g + 1) * _SLAB],
            (((1,), (0,)), ((), ())), preferred_element_type=jnp.float32)
        d0 = (h2b + c2_ref[:, g * _SLAB:g * _SLAB + 128]) + mm2[:, :128]
        d1 = (h2b + c2_ref[:, g * _SLAB + 128:(g + 1) * _SLAB]) + mm2[:, 128:]
        take1 = d1 < d0
        sval = jnp.where(take1, d1, d0)
        sidx = jnp.where(take1, 2 * g + 1, 2 * g)
        if val is None:
            val, gidx = sval, sidx
        else:
            upd = sval < val
            val = jnp.where(upd, sval, val)
            gidx = jnp.where(upd, sidx, gidx)

    col = gidx * 128 + lax.broadcasted_iota(jnp.int32, (_M_BLK, 128), 1)
    best = jnp.min(val, axis=1, keepdims=True)
    idx = jnp.min(jnp.where(val == best, col, _CB), axis=1, keepdims=True)
    idx_ref[...] = idx

    @pl.when(m == 0)
    def _zero():
        dsum_ref[...] = jnp.zeros_like(dsum_ref)

    dsum_ref[...] += jnp.sum(best, keepdims=True)


def _argmin_call(flat, codebook):
    m_steps = flat.shape[0] // _M_BLK
    return pl.pallas_call(
        _argmin_body,
        grid=(m_steps,),
        in_specs=[
            pl.BlockSpec((_M_BLK, _D), lambda m: (m, 0)),
            pl.BlockSpec((_D, _CB), lambda m: (0, 0)),
        ],
        out_specs=[
            pl.BlockSpec((_M_BLK, 1), lambda m: (m, 0)),
            pl.BlockSpec((1, 1), lambda m: (0, 0)),
        ],
        out_shape=[
            jax.ShapeDtypeStruct((flat.shape[0], 1), jnp.int32),
            jax.ShapeDtypeStruct((1, 1), jnp.float32),
        ],
        scratch_shapes=[
            pltpu.VMEM((1, _CB), jnp.float32),
        ],
        compiler_params=pltpu.CompilerParams(
            dimension_semantics=("arbitrary",),
        ),
    )(flat, codebook.T)


def _gather_z(codebook, idx_flat):
    info = plsc.get_sparse_core_info()
    nc, ns = info.num_cores, info.num_subcores
    nw = nc * ns
    b = idx_flat.shape[0]
    b_per_w = b // nw
    chunk = 128
    n_chunks = b_per_w // chunk
    idx2d = idx_flat.reshape(nw * n_chunks, chunk)
    mesh = plsc.VectorSubcoreMesh(core_axis_name="c", subcore_axis_name="s")

    @functools.partial(
        pl.kernel,
        out_type=jax.ShapeDtypeStruct((b, _D), jnp.float32),
        mesh=mesh,
        scratch_types=[
            pltpu.VMEM((n_chunks, chunk), jnp.int32),
            pltpu.VMEM((chunk, _D), jnp.float32),
            pltpu.SemaphoreType.DMA,
        ],
    )
    def gather_k(cb_hbm, idx_hbm, out_hbm, idx_v, rows_v, sem):
        wid = lax.axis_index("s") * nc + lax.axis_index("c")
        pltpu.sync_copy(idx_hbm.at[pl.ds(wid * n_chunks, n_chunks)], idx_v)
        for cc in range(n_chunks):
            pltpu.async_copy(cb_hbm.at[idx_v.at[cc]], rows_v, sem).wait()
            pltpu.sync_copy(
                rows_v, out_hbm.at[pl.ds(wid * b_per_w + cc * chunk, chunk)])

    return gather_k(codebook, idx2d)


def kernel(h, codebook):
    b, t, d = h.shape
    flat = h.reshape(-1, d)
    idx_col, dsum = _argmin_call(flat, codebook)
    idx_flat = idx_col.reshape(-1)
    z = _gather_z(codebook, idx_flat)
    z_q = z.reshape(b, t, d)
    loss = dsum[0, 0] / (b * t * d)
    indices = idx_flat.reshape(b, t)
    return z_q, indices, loss, loss


# speedup vs baseline: 2.3646x; 1.0558x over previous
import functools

import jax
import jax.numpy as jnp
from jax import lax
from jax.experimental import pallas as pl
from jax.experimental.pallas import tpu as pltpu
from jax.experimental.pallas import tpu_sc as plsc

_CB = 8192
_D = 256
_M_BLK = 2048
_SLAB = 256
_N_SLABS = _CB // _SLAB


def _argmin_body(h_ref, cb_ref, idx_ref, dsum_ref, c2_ref, cbt_ref):
    m = pl.program_id(0)

    @pl.when(m == 0)
    def _precompute():
        cbt_ref[...] = cb_ref[...].T
        c = cbt_ref[...]
        c2_ref[...] = jnp.sum(c * c, axis=0)[None, :]

    hm = h_ref[...]
    hm2 = hm * (-2.0)
    h2 = jnp.sum(hm * hm, axis=1, keepdims=True)
    h2b = jnp.broadcast_to(h2, (_M_BLK, 128))

    val = None
    gidx = None
    for g in range(_N_SLABS):
        mm2 = lax.dot_general(
            hm2, cbt_ref[:, g * _SLAB:(g + 1) * _SLAB],
            (((1,), (0,)), ((), ())), preferred_element_type=jnp.float32)
        d0 = (h2b + c2_ref[:, g * _SLAB:g * _SLAB + 128]) + mm2[:, :128]
        d1 = (h2b + c2_ref[:, g * _SLAB + 128:(g + 1) * _SLAB]) + mm2[:, 128:]
        take1 = d1 < d0
        sval = jnp.where(take1, d1, d0)
        sidx = jnp.where(take1, 2 * g + 1, 2 * g)
        if val is None:
            val, gidx = sval, sidx
        else:
            upd = sval < val
            val = jnp.where(upd, sval, val)
            gidx = jnp.where(upd, sidx, gidx)

    col = gidx * 128 + lax.broadcasted_iota(jnp.int32, (_M_BLK, 128), 1)
    best = jnp.min(val, axis=1, keepdims=True)
    idx = jnp.min(jnp.where(val == best, col, _CB), axis=1, keepdims=True)
    idx_ref[...] = idx

    @pl.when(m == 0)
    def _zero():
        dsum_ref[...] = jnp.zeros_like(dsum_ref)

    dsum_ref[...] += jnp.sum(best, keepdims=True)


def _argmin_call(flat, codebook):
    m_steps = flat.shape[0] // _M_BLK
    return pl.pallas_call(
        _argmin_body,
        grid=(m_steps,),
        in_specs=[
            pl.BlockSpec((_M_BLK, _D), lambda m: (m, 0)),
            pl.BlockSpec((_CB, _D), lambda m: (0, 0)),
        ],
        out_specs=[
            pl.BlockSpec((_M_BLK, 1), lambda m: (m, 0)),
            pl.BlockSpec((1, 1), lambda m: (0, 0)),
        ],
        out_shape=[
            jax.ShapeDtypeStruct((flat.shape[0], 1), jnp.int32),
            jax.ShapeDtypeStruct((1, 1), jnp.float32),
        ],
        scratch_shapes=[
            pltpu.VMEM((1, _CB), jnp.float32),
            pltpu.VMEM((_D, _CB), jnp.float32),
        ],
        compiler_params=pltpu.CompilerParams(
            dimension_semantics=("arbitrary",),
        ),
    )(flat, codebook)


def _gather_z(codebook, idx_flat):
    info = plsc.get_sparse_core_info()
    nc, ns = info.num_cores, info.num_subcores
    nw = nc * ns
    b = idx_flat.shape[0]
    b_per_w = b // nw
    chunk = 128
    n_chunks = b_per_w // chunk
    idx2d = idx_flat.reshape(nw * n_chunks, chunk)
    mesh = plsc.VectorSubcoreMesh(core_axis_name="c", subcore_axis_name="s")

    @functools.partial(
        pl.kernel,
        out_type=jax.ShapeDtypeStruct((b, _D), jnp.float32),
        mesh=mesh,
        scratch_types=[
            pltpu.VMEM((n_chunks, chunk), jnp.int32),
            pltpu.VMEM((chunk, _D), jnp.float32),
            pltpu.VMEM((chunk, _D), jnp.float32),
            pltpu.SemaphoreType.DMA,
            pltpu.SemaphoreType.DMA,
        ],
    )
    def gather_k(cb_hbm, idx_hbm, out_hbm, idx_v, rows_a, rows_b, sem_a,
                 sem_b):
        wid = lax.axis_index("s") * nc + lax.axis_index("c")
        pltpu.sync_copy(idx_hbm.at[pl.ds(wid * n_chunks, n_chunks)], idx_v)
        bufs = (rows_a, rows_b)
        sems = (sem_a, sem_b)
        pend = [None, None]
        pend[0] = pltpu.async_copy(cb_hbm.at[idx_v.at[0]], rows_a, sem_a)
        for cc in range(n_chunks):
            if cc + 1 < n_chunks:
                pend[(cc + 1) % 2] = pltpu.async_copy(
                    cb_hbm.at[idx_v.at[cc + 1]], bufs[(cc + 1) % 2],
                    sems[(cc + 1) % 2])
            pend[cc % 2].wait()
            pltpu.sync_copy(
                bufs[cc % 2],
                out_hbm.at[pl.ds(wid * b_per_w + cc * chunk, chunk)])

    return gather_k(codebook, idx2d)


def kernel(h, codebook):
    b, t, d = h.shape
    flat = h.reshape(-1, d)
    idx_col, dsum = _argmin_call(flat, codebook)
    idx_flat = idx_col.reshape(-1)
    z = _gather_z(codebook, idx_flat)
    z_q = z.reshape(b, t, d)
    loss = dsum[0, 0] / (b * t * d)
    indices = idx_flat.reshape(b, t)
    return z_q, indices, loss, loss
